# baseline (speedup 1.0000x reference)
import functools

import jax
import jax.numpy as jnp
from jax import lax
from jax.experimental import pallas as pl
from jax.experimental.pallas import tpu as pltpu
from jax.experimental.pallas import tpu_sc as plsc

NV = 1000
NP = 1024
D = 128
NW = 32
CHUNK = 128
NVOX = 16 * 32 * 32 * 32
BLK = NVOX // NW
HALO = 1024
WINDOWS = BLK // CHUNK
CBINS = NP * NP
SLICE = CBINS // 16
ABLK = 8


def _hist_body(s_hbm, z_hbm, out_hbm, blk_v, ia_v, ib_v, ones_v, c_sh,
               sem_a, sem_b):
    cid = lax.axis_index("c")
    sid = lax.axis_index("s")
    wid = sid * 2 + cid
    pltpu.sync_copy(z_hbm, c_sh.at[pl.ds(sid * SLICE, SLICE)])
    pltpu.sync_copy(s_hbm.at[pl.ds(wid * BLK, BLK)], blk_v.at[pl.ds(0, BLK)])

    @pl.when(wid < NW - 1)
    def _():
        pltpu.sync_copy(s_hbm.at[pl.ds(wid * BLK + BLK, HALO)],
                        blk_v.at[pl.ds(BLK, HALO)])

    for k in range(CHUNK // 16):
        ones_v[pl.ds(k * 16, 16)] = jnp.full((16,), 1.0, jnp.float32)
    xlim = jnp.where(wid % 2 == 0, 16, 15)
    lane = lax.iota(jnp.int32, 16)
    plsc.subcore_barrier()

    def compute(j, buf):
        off = j * CHUNK
        for k in range(CHUNK // 16):
            o = off + k * 16
            v0 = blk_v[pl.ds(o, 16)]
            v1 = blk_v[pl.ds(o + 1, 16)]
            v32 = blk_v[pl.ds(o + 32, 16)]
            v1024 = blk_v[pl.ds(o + 1024, 16)]
            pos = lane + o
            zok = (pos & 31) != 31
            yok = ((pos >> 5) & 31) != 31
            xok = (pos >> 10) < xlim
            buf[0, pl.ds(k * 16, 16)] = jnp.where(
                zok, jnp.minimum(v0, v1) * NP + jnp.maximum(v0, v1), 0)
            buf[1, pl.ds(k * 16, 16)] = jnp.where(
                yok, jnp.minimum(v0, v32) * NP + jnp.maximum(v0, v32), 0)
            buf[2, pl.ds(k * 16, 16)] = jnp.where(
                xok, jnp.minimum(v0, v1024) * NP + jnp.maximum(v0, v1024), 0)

    def fire(buf, sem):
        for r in range(3):
            pltpu.async_copy(ones_v, c_sh.at[buf.at[r]], sem, add=True)

    def drain(buf, sem):
        for r in range(3):
            pltpu.make_async_copy(ones_v, c_sh.at[buf.at[r]], sem).wait()

    def body(jj, carry):
        compute(2 * jj, ia_v)

        @pl.when(jj > 0)
        def _():
            drain(ib_v, sem_b)

        fire(ia_v, sem_a)
        compute(2 * jj + 1, ib_v)
        drain(ia_v, sem_a)
        fire(ib_v, sem_b)
        return carry

    lax.fori_loop(0, WINDOWS // 2, body, 0)
    drain(ib_v, sem_b)
    plsc.subcore_barrier()
    pltpu.sync_copy(c_sh.at[pl.ds(sid * SLICE, SLICE)],
                    out_hbm.at[pl.ds(cid * CBINS + sid * SLICE, SLICE)])


@functools.cache
def _hist():
    return functools.partial(
        pl.kernel,
        mesh=plsc.VectorSubcoreMesh(core_axis_name="c", subcore_axis_name="s",
                                    num_cores=2, num_subcores=16),
        out_type=jax.ShapeDtypeStruct((2 * CBINS,), jnp.float32),
        scratch_types=[
            pltpu.VMEM((BLK + HALO,), jnp.int32),
            pltpu.VMEM((3, CHUNK), jnp.int32),
            pltpu.VMEM((3, CHUNK), jnp.int32),
            pltpu.VMEM((CHUNK,), jnp.float32),
            pltpu.VMEM_SHARED((CBINS,), jnp.float32),
            pltpu.SemaphoreType.DMA,
            pltpu.SemaphoreType.DMA,
        ],
    )(_hist_body)


KB = 256


def _dmat_body(et_ref, ea_ref, d_ref, cols_ref):
    i = pl.program_id(0)
    k = pl.program_id(1)

    @pl.when(k == 0)
    def _():
        for j in range(ABLK):
            cols_ref[j] = jnp.transpose(
                ea_ref[pl.ds(j, 1), :]).astype(jnp.bfloat16)

    active = (k + 1) * KB > i * ABLK

    @pl.when(active)
    def _():
        rows = []
        for j in range(ABLK):
            x = jnp.abs(et_ref[...] - cols_ref[j])
            h = D
            while h > 16:
                h //= 2
                x = x[:h] + x[h:]
            rows.append(jnp.sum(x, axis=0, keepdims=True))
        d_ref[...] = jnp.concatenate(rows, axis=0)

    @pl.when(jnp.logical_not(active))
    def _():
        d_ref[...] = jnp.zeros((ABLK, KB), jnp.bfloat16)


def _dmat(et, ep):
    return pl.pallas_call(
        _dmat_body,
        grid=(NP // ABLK, NP // KB),
        in_specs=[
            pl.BlockSpec((D, KB), lambda i, k: (0, k)),
            pl.BlockSpec((ABLK, D), lambda i, k: (i, 0)),
        ],
        out_specs=pl.BlockSpec((ABLK, KB), lambda i, k: (i, k)),
        out_shape=jax.ShapeDtypeStruct((NP, NP), jnp.bfloat16),
        scratch_shapes=[pltpu.VMEM((ABLK, D, 1), jnp.bfloat16)],
    )(et, ep)


RBLK = 128


def _contract_body(c_ref, d_ref, out_ref, acc_ref):
    i = pl.program_id(0)
    contrib = jnp.sum((c_ref[0] + c_ref[1]) * d_ref[...].astype(jnp.float32),
                      axis=0, keepdims=True)

    @pl.when(i == 0)
    def _():
        acc_ref[...] = contrib

    @pl.when(i > 0)
    def _():
        acc_ref[...] = acc_ref[...] + contrib

    @pl.when(i == pl.num_programs(0) - 1)
    def _():
        out_ref[0, 0] = jnp.sum(acc_ref[...])


def _contract(c2, dm):
    return pl.pallas_call(
        _contract_body,
        grid=(NP // RBLK,),
        in_specs=[
            pl.BlockSpec((2, RBLK, NP), lambda i: (0, i, 0)),
            pl.BlockSpec((RBLK, NP), lambda i: (i, 0)),
        ],
        out_specs=pl.BlockSpec((1, 1), lambda i: (0, 0),
                               memory_space=pltpu.SMEM),
        out_shape=jax.ShapeDtypeStruct((1, 1), jnp.float32),
        scratch_shapes=[pltpu.VMEM((1, NP), jnp.float32)],
    )(c2, dm)


@jax.jit
def kernel(structure, embeddings):
    s = structure
    zeros = jnp.zeros((SLICE,), jnp.float32)
    c2 = _hist()(s.ravel(), zeros).reshape(2, NP, NP)
    ep = jnp.zeros((NP, D), jnp.float32).at[:NV].set(embeddings)
    dm = _dmat(ep.T.astype(jnp.bfloat16), ep)
    total = _contract(c2, dm)[0, 0]
    n_pairs = s.shape[0] * (s.shape[1] - 1) * s.shape[2] * s.shape[3]
    return total / (3.0 * n_pairs * D)


# speedup vs baseline: 3.0839x; 3.0839x over previous
import functools

import jax
import jax.numpy as jnp
from jax import lax
from jax.experimental import pallas as pl
from jax.experimental.pallas import tpu as pltpu
from jax.experimental.pallas import tpu_sc as plsc

NV = 1000
NP = 1024
D = 128
NW = 32
CHUNK = 128
NVOX = 16 * 32 * 32 * 32
BLK = NVOX // NW
HALO = 1024
WINDOWS = BLK // CHUNK
CBINS = NP * NP
SLICE = CBINS // 16
ABLK = 16


def _hist_body(s_hbm, z_hbm, out_hbm, blk_v, ia_v, ib_v, ones_v, c_sh,
               sem_a, sem_b):
    cid = lax.axis_index("c")
    sid = lax.axis_index("s")
    wid = sid * 2 + cid
    pltpu.sync_copy(z_hbm, c_sh.at[pl.ds(sid * SLICE, SLICE)])
    pltpu.sync_copy(s_hbm.at[pl.ds(wid * BLK, BLK)], blk_v.at[pl.ds(0, BLK)])

    @pl.when(wid < NW - 1)
    def _():
        pltpu.sync_copy(s_hbm.at[pl.ds(wid * BLK + BLK, HALO)],
                        blk_v.at[pl.ds(BLK, HALO)])

    for k in range(CHUNK // 16):
        ones_v[pl.ds(k * 16, 16)] = jnp.full((16,), 1.0, jnp.float32)
    xlim = jnp.where(wid % 2 == 0, 16, 15)
    lane = lax.iota(jnp.int32, 16)
    plsc.subcore_barrier()

    def compute(j, buf):
        off = j * CHUNK
        for k in range(CHUNK // 16):
            o = off + k * 16
            v0 = blk_v[pl.ds(o, 16)]
            v1 = blk_v[pl.ds(o + 1, 16)]
            v32 = blk_v[pl.ds(o + 32, 16)]
            v1024 = blk_v[pl.ds(o + 1024, 16)]
            pos = lane + o
            zok = (pos & 31) != 31
            yok = ((pos >> 5) & 31) != 31
            xok = (pos >> 10) < xlim
            buf[0, pl.ds(k * 16, 16)] = jnp.where(zok, v0 * NP + v1, 0)
            buf[1, pl.ds(k * 16, 16)] = jnp.where(yok, v0 * NP + v32, 0)
            buf[2, pl.ds(k * 16, 16)] = jnp.where(xok, v0 * NP + v1024, 0)

    def fire(buf, sem):
        for r in range(3):
            pltpu.async_copy(ones_v, c_sh.at[buf.at[r]], sem, add=True)

    def drain(buf, sem):
        for r in range(3):
            pltpu.make_async_copy(ones_v, c_sh.at[buf.at[r]], sem).wait()

    def body(jj, carry):
        compute(2 * jj, ia_v)

        @pl.when(jj > 0)
        def _():
            drain(ib_v, sem_b)

        fire(ia_v, sem_a)
        compute(2 * jj + 1, ib_v)
        drain(ia_v, sem_a)
        fire(ib_v, sem_b)
        return carry

    lax.fori_loop(0, WINDOWS // 2, body, 0)
    drain(ib_v, sem_b)
    plsc.subcore_barrier()
    pltpu.sync_copy(c_sh.at[pl.ds(sid * SLICE, SLICE)],
                    out_hbm.at[pl.ds(cid * CBINS + sid * SLICE, SLICE)])


@functools.cache
def _hist():
    return functools.partial(
        pl.kernel,
        mesh=plsc.VectorSubcoreMesh(core_axis_name="c", subcore_axis_name="s",
                                    num_cores=2, num_subcores=16),
        out_type=jax.ShapeDtypeStruct((2 * CBINS,), jnp.float32),
        scratch_types=[
            pltpu.VMEM((BLK + HALO,), jnp.int32),
            pltpu.VMEM((3, CHUNK), jnp.int32),
            pltpu.VMEM((3, CHUNK), jnp.int32),
            pltpu.VMEM((CHUNK,), jnp.float32),
            pltpu.VMEM_SHARED((CBINS,), jnp.float32),
            pltpu.SemaphoreType.DMA,
            pltpu.SemaphoreType.DMA,
        ],
    )(_hist_body)


def _dmat_body(et_ref, ea_ref, d_ref):
    rows = []
    for j in range(ABLK):
        col = jnp.transpose(ea_ref[pl.ds(j, 1), :])
        x = jnp.abs(et_ref[...] - col.astype(jnp.bfloat16))
        h = D
        while h > 16:
            h //= 2
            x = x[:h] + x[h:]
        dj = jnp.sum(x, axis=0, keepdims=True)
        rows.append(dj)
    d_ref[...] = jnp.concatenate(rows, axis=0)


def _dmat(et, ep):
    return pl.pallas_call(
        _dmat_body,
        grid=(NP // ABLK,),
        in_specs=[
            pl.BlockSpec((D, NP), lambda i: (0, 0)),
            pl.BlockSpec((ABLK, D), lambda i: (i, 0)),
        ],
        out_specs=pl.BlockSpec((ABLK, NP), lambda i: (i, 0)),
        out_shape=jax.ShapeDtypeStruct((NP, NP), jnp.bfloat16),
    )(et, ep)


RBLK = 128


def _contract_body(c_ref, d_ref, out_ref, acc_ref):
    i = pl.program_id(0)
    contrib = jnp.sum((c_ref[0] + c_ref[1]) * d_ref[...].astype(jnp.float32),
                      axis=0, keepdims=True)

    @pl.when(i == 0)
    def _():
        acc_ref[...] = contrib

    @pl.when(i > 0)
    def _():
        acc_ref[...] = acc_ref[...] + contrib

    @pl.when(i == pl.num_programs(0) - 1)
    def _():
        out_ref[0, 0] = jnp.sum(acc_ref[...])


def _contract(c2, dm):
    return pl.pallas_call(
        _contract_body,
        grid=(NP // RBLK,),
        in_specs=[
            pl.BlockSpec((2, RBLK, NP), lambda i: (0, i, 0)),
            pl.BlockSpec((RBLK, NP), lambda i: (i, 0)),
        ],
        out_specs=pl.BlockSpec((1, 1), lambda i: (0, 0),
                               memory_space=pltpu.SMEM),
        out_shape=jax.ShapeDtypeStruct((1, 1), jnp.float32),
        scratch_shapes=[pltpu.VMEM((1, NP), jnp.float32)],
    )(c2, dm)


@jax.jit
def kernel(structure, embeddings):
    s = structure
    zeros = jnp.zeros((SLICE,), jnp.float32)
    c2 = _hist()(s.ravel(), zeros).reshape(2, NP, NP)
    ep = jnp.zeros((NP, D), jnp.float32).at[:NV].set(embeddings)
    dm = _dmat(ep.T.astype(jnp.bfloat16), ep)
    total = _contract(c2, dm)[0, 0]
    n_pairs = s.shape[0] * (s.shape[1] - 1) * s.shape[2] * s.shape[3]
    return total / (3.0 * n_pairs * D)


# speedup vs baseline: 3.2007x; 1.0378x over previous
import functools

import jax
import jax.numpy as jnp
from jax import lax
from jax.experimental import pallas as pl
from jax.experimental.pallas import tpu as pltpu
from jax.experimental.pallas import tpu_sc as plsc

NV = 1000
NP = 1024
D = 128
NW = 32
CHUNK = 128
NVOX = 16 * 32 * 32 * 32
BLK = NVOX // NW
HALO = 1024
WINDOWS = BLK // CHUNK
CBINS = NP * NP
SLICE = CBINS // 16
ABLK = 32


def _hist_body(s_hbm, z_hbm, out_hbm, blk_v, ia_v, ib_v, ones_v, c_sh,
               sem_a, sem_b):
    cid = lax.axis_index("c")
    sid = lax.axis_index("s")
    wid = sid * 2 + cid
    pltpu.sync_copy(z_hbm, c_sh.at[pl.ds(sid * SLICE, SLICE)])
    pltpu.sync_copy(s_hbm.at[pl.ds(wid * BLK, BLK)], blk_v.at[pl.ds(0, BLK)])

    @pl.when(wid < NW - 1)
    def _():
        pltpu.sync_copy(s_hbm.at[pl.ds(wid * BLK + BLK, HALO)],
                        blk_v.at[pl.ds(BLK, HALO)])

    for k in range(CHUNK // 16):
        ones_v[pl.ds(k * 16, 16)] = jnp.full((16,), 1.0, jnp.float32)
    xlim = jnp.where(wid % 2 == 0, 16, 15)
    lane = lax.iota(jnp.int32, 16)
    plsc.subcore_barrier()

    def compute(j, buf):
        off = j * CHUNK
        for k in range(CHUNK // 16):
            o = off + k * 16
            v0 = blk_v[pl.ds(o, 16)]
            v1 = blk_v[pl.ds(o + 1, 16)]
            v32 = blk_v[pl.ds(o + 32, 16)]
            v1024 = blk_v[pl.ds(o + 1024, 16)]
            pos = lane + o
            zok = (pos & 31) != 31
            yok = ((pos >> 5) & 31) != 31
            xok = (pos >> 10) < xlim
            buf[0, pl.ds(k * 16, 16)] = jnp.where(zok, v0 * NP + v1, 0)
            buf[1, pl.ds(k * 16, 16)] = jnp.where(yok, v0 * NP + v32, 0)
            buf[2, pl.ds(k * 16, 16)] = jnp.where(xok, v0 * NP + v1024, 0)

    def fire(buf, sem):
        for r in range(3):
            pltpu.async_copy(ones_v, c_sh.at[buf.at[r]], sem, add=True)

    def drain(buf, sem):
        for r in range(3):
            pltpu.make_async_copy(ones_v, c_sh.at[buf.at[r]], sem).wait()

    def body(jj, carry):
        compute(2 * jj, ia_v)

        @pl.when(jj > 0)
        def _():
            drain(ib_v, sem_b)

        fire(ia_v, sem_a)
        compute(2 * jj + 1, ib_v)
        drain(ia_v, sem_a)
        fire(ib_v, sem_b)
        return carry

    lax.fori_loop(0, WINDOWS // 2, body, 0)
    drain(ib_v, sem_b)
    plsc.subcore_barrier()
    pltpu.sync_copy(c_sh.at[pl.ds(sid * SLICE, SLICE)],
                    out_hbm.at[pl.ds(cid * CBINS + sid * SLICE, SLICE)])


@functools.cache
def _hist():
    return functools.partial(
        pl.kernel,
        mesh=plsc.VectorSubcoreMesh(core_axis_name="c", subcore_axis_name="s",
                                    num_cores=2, num_subcores=16),
        out_type=jax.ShapeDtypeStruct((2 * CBINS,), jnp.float32),
        scratch_types=[
            pltpu.VMEM((BLK + HALO,), jnp.int32),
            pltpu.VMEM((3, CHUNK), jnp.int32),
            pltpu.VMEM((3, CHUNK), jnp.int32),
            pltpu.VMEM((CHUNK,), jnp.float32),
            pltpu.VMEM_SHARED((CBINS,), jnp.float32),
            pltpu.SemaphoreType.DMA,
            pltpu.SemaphoreType.DMA,
        ],
    )(_hist_body)


def _dmat_body(et_ref, ea_ref, d_ref):
    rows = []
    for j in range(ABLK):
        col = jnp.transpose(ea_ref[pl.ds(j, 1), :])
        x = jnp.abs(et_ref[...] - col.astype(jnp.bfloat16))
        h = D
        while h > 16:
            h //= 2
            x = x[:h] + x[h:]
        dj = jnp.sum(x, axis=0, keepdims=True)
        rows.append(dj)
    d_ref[...] = jnp.concatenate(rows, axis=0)


def _dmat(et, ep):
    return pl.pallas_call(
        _dmat_body,
        grid=(NP // ABLK,),
        in_specs=[
            pl.BlockSpec((D, NP), lambda i: (0, 0)),
            pl.BlockSpec((ABLK, D), lambda i: (i, 0)),
        ],
        out_specs=pl.BlockSpec((ABLK, NP), lambda i: (i, 0)),
        out_shape=jax.ShapeDtypeStruct((NP, NP), jnp.bfloat16),
    )(et, ep)


RBLK = 128


def _contract_body(c_ref, d_ref, out_ref, acc_ref):
    i = pl.program_id(0)
    contrib = jnp.sum((c_ref[0] + c_ref[1]) * d_ref[...].astype(jnp.float32),
                      axis=0, keepdims=True)

    @pl.when(i == 0)
    def _():
        acc_ref[...] = contrib

    @pl.when(i > 0)
    def _():
        acc_ref[...] = acc_ref[...] + contrib

    @pl.when(i == pl.num_programs(0) - 1)
    def _():
        out_ref[0, 0] = jnp.sum(acc_ref[...])


def _contract(c2, dm):
    return pl.pallas_call(
        _contract_body,
        grid=(NP // RBLK,),
        in_specs=[
            pl.BlockSpec((2, RBLK, NP), lambda i: (0, i, 0)),
            pl.BlockSpec((RBLK, NP), lambda i: (i, 0)),
        ],
        out_specs=pl.BlockSpec((1, 1), lambda i: (0, 0),
                               memory_space=pltpu.SMEM),
        out_shape=jax.ShapeDtypeStruct((1, 1), jnp.float32),
        scratch_shapes=[pltpu.VMEM((1, NP), jnp.float32)],
    )(c2, dm)


@jax.jit
def kernel(structure, embeddings):
    s = structure
    zeros = jnp.zeros((SLICE,), jnp.float32)
    c2 = _hist()(s.ravel(), zeros).reshape(2, NP, NP)
    ep = jnp.zeros((NP, D), jnp.float32).at[:NV].set(embeddings)
    dm = _dmat(ep.T.astype(jnp.bfloat16), ep)
    total = _contract(c2, dm)[0, 0]
    n_pairs = s.shape[0] * (s.shape[1] - 1) * s.shape[2] * s.shape[3]
    return total / (3.0 * n_pairs * D)


# speedup vs baseline: 3.2358x; 1.0110x over previous
import functools

import jax
import jax.numpy as jnp
from jax import lax
from jax.experimental import pallas as pl
from jax.experimental.pallas import tpu as pltpu
from jax.experimental.pallas import tpu_sc as plsc

NV = 1000
NP = 1024
D = 128
NW = 32
CHUNK = 128
NVOX = 16 * 32 * 32 * 32
BLK = NVOX // NW
HALO = 1024
WINDOWS = BLK // CHUNK
CBINS = NP * NP
SLICE = CBINS // 16
ABLK = 64


def _hist_body(s_hbm, z_hbm, out_hbm, blk_v, ia_v, ib_v, ones_v, c_sh,
               sem_a, sem_b):
    cid = lax.axis_index("c")
    sid = lax.axis_index("s")
    wid = sid * 2 + cid
    pltpu.sync_copy(z_hbm, c_sh.at[pl.ds(sid * SLICE, SLICE)])
    pltpu.sync_copy(s_hbm.at[pl.ds(wid * BLK, BLK)], blk_v.at[pl.ds(0, BLK)])

    @pl.when(wid < NW - 1)
    def _():
        pltpu.sync_copy(s_hbm.at[pl.ds(wid * BLK + BLK, HALO)],
                        blk_v.at[pl.ds(BLK, HALO)])

    for k in range(CHUNK // 16):
        ones_v[pl.ds(k * 16, 16)] = jnp.full((16,), 1.0, jnp.float32)
    xlim = jnp.where(wid % 2 == 0, 16, 15)
    lane = lax.iota(jnp.int32, 16)
    plsc.subcore_barrier()

    def compute(j, buf):
        off = j * CHUNK
        for k in range(CHUNK // 16):
            o = off + k * 16
            v0 = blk_v[pl.ds(o, 16)]
            v1 = blk_v[pl.ds(o + 1, 16)]
            v32 = blk_v[pl.ds(o + 32, 16)]
            v1024 = blk_v[pl.ds(o + 1024, 16)]
            pos = lane + o
            zok = (pos & 31) != 31
            yok = ((pos >> 5) & 31) != 31
            xok = (pos >> 10) < xlim
            buf[0, pl.ds(k * 16, 16)] = jnp.where(zok, v0 * NP + v1, 0)
            buf[1, pl.ds(k * 16, 16)] = jnp.where(yok, v0 * NP + v32, 0)
            buf[2, pl.ds(k * 16, 16)] = jnp.where(xok, v0 * NP + v1024, 0)

    def fire(buf, sem):
        for r in range(3):
            pltpu.async_copy(ones_v, c_sh.at[buf.at[r]], sem, add=True)

    def drain(buf, sem):
        for r in range(3):
            pltpu.make_async_copy(ones_v, c_sh.at[buf.at[r]], sem).wait()

    def body(jj, carry):
        compute(2 * jj, ia_v)

        @pl.when(jj > 0)
        def _():
            drain(ib_v, sem_b)

        fire(ia_v, sem_a)
        compute(2 * jj + 1, ib_v)
        drain(ia_v, sem_a)
        fire(ib_v, sem_b)
        return carry

    lax.fori_loop(0, WINDOWS // 2, body, 0)
    drain(ib_v, sem_b)
    plsc.subcore_barrier()
    pltpu.sync_copy(c_sh.at[pl.ds(sid * SLICE, SLICE)],
                    out_hbm.at[pl.ds(cid * CBINS + sid * SLICE, SLICE)])


@functools.cache
def _hist():
    return functools.partial(
        pl.kernel,
        mesh=plsc.VectorSubcoreMesh(core_axis_name="c", subcore_axis_name="s",
                                    num_cores=2, num_subcores=16),
        out_type=jax.ShapeDtypeStruct((2 * CBINS,), jnp.float32),
        scratch_types=[
            pltpu.VMEM((BLK + HALO,), jnp.int32),
            pltpu.VMEM((3, CHUNK), jnp.int32),
            pltpu.VMEM((3, CHUNK), jnp.int32),
            pltpu.VMEM((CHUNK,), jnp.float32),
            pltpu.VMEM_SHARED((CBINS,), jnp.float32),
            pltpu.SemaphoreType.DMA,
            pltpu.SemaphoreType.DMA,
        ],
    )(_hist_body)


def _dmat_body(et_ref, ea_ref, d_ref):
    rows = []
    for j in range(ABLK):
        col = jnp.transpose(ea_ref[pl.ds(j, 1), :])
        x = jnp.abs(et_ref[...] - col.astype(jnp.bfloat16))
        h = D
        while h > 16:
            h //= 2
            x = x[:h] + x[h:]
        dj = jnp.sum(x, axis=0, keepdims=True)
        rows.append(dj)
    d_ref[...] = jnp.concatenate(rows, axis=0)


def _dmat(et, ep):
    return pl.pallas_call(
        _dmat_body,
        grid=(NP // ABLK,),
        in_specs=[
            pl.BlockSpec((D, NP), lambda i: (0, 0)),
            pl.BlockSpec((ABLK, D), lambda i: (i, 0)),
        ],
        out_specs=pl.BlockSpec((ABLK, NP), lambda i: (i, 0)),
        out_shape=jax.ShapeDtypeStruct((NP, NP), jnp.bfloat16),
    )(et, ep)


RBLK = 128


def _contract_body(c_ref, d_ref, out_ref, acc_ref):
    i = pl.program_id(0)
    contrib = jnp.sum((c_ref[0] + c_ref[1]) * d_ref[...].astype(jnp.float32),
                      axis=0, keepdims=True)

    @pl.when(i == 0)
    def _():
        acc_ref[...] = contrib

    @pl.when(i > 0)
    def _():
        acc_ref[...] = acc_ref[...] + contrib

    @pl.when(i == pl.num_programs(0) - 1)
    def _():
        out_ref[0, 0] = jnp.sum(acc_ref[...])


def _contract(c2, dm):
    return pl.pallas_call(
        _contract_body,
        grid=(NP // RBLK,),
        in_specs=[
            pl.BlockSpec((2, RBLK, NP), lambda i: (0, i, 0)),
            pl.BlockSpec((RBLK, NP), lambda i: (i, 0)),
        ],
        out_specs=pl.BlockSpec((1, 1), lambda i: (0, 0),
                               memory_space=pltpu.SMEM),
        out_shape=jax.ShapeDtypeStruct((1, 1), jnp.float32),
        scratch_shapes=[pltpu.VMEM((1, NP), jnp.float32)],
    )(c2, dm)


@jax.jit
def kernel(structure, embeddings):
    s = structure
    zeros = jnp.zeros((SLICE,), jnp.float32)
    c2 = _hist()(s.ravel(), zeros).reshape(2, NP, NP)
    ep = jnp.zeros((NP, D), jnp.float32).at[:NV].set(embeddings)
    dm = _dmat(ep.T.astype(jnp.bfloat16), ep)
    total = _contract(c2, dm)[0, 0]
    n_pairs = s.shape[0] * (s.shape[1] - 1) * s.shape[2] * s.shape[3]
    return total / (3.0 * n_pairs * D)


# speedup vs baseline: 3.3243x; 1.0273x over previous
import functools

import jax
import jax.numpy as jnp
from jax import lax
from jax.experimental import pallas as pl
from jax.experimental.pallas import tpu as pltpu
from jax.experimental.pallas import tpu_sc as plsc

NV = 1000
NP = 1024
D = 128
NW = 32
CHUNK = 128
NVOX = 16 * 32 * 32 * 32
BLK = NVOX // NW
HALO = 1024
WINDOWS = BLK // CHUNK
CBINS = NP * NP
SLICE = CBINS // 16
ABLK = 64


def _hist_body(s_hbm, z_hbm, out_hbm, blk_v, ia_v, ib_v, ones_v, c_sh,
               sem_a, sem_b):
    cid = lax.axis_index("c")
    sid = lax.axis_index("s")
    wid = sid * 2 + cid
    pltpu.sync_copy(z_hbm, c_sh.at[pl.ds(sid * SLICE, SLICE)])
    pltpu.sync_copy(s_hbm.at[pl.ds(wid * BLK, BLK)], blk_v.at[pl.ds(0, BLK)])

    @pl.when(wid < NW - 1)
    def _():
        pltpu.sync_copy(s_hbm.at[pl.ds(wid * BLK + BLK, HALO)],
                        blk_v.at[pl.ds(BLK, HALO)])

    for k in range(CHUNK // 16):
        ones_v[pl.ds(k * 16, 16)] = jnp.full((16,), 1.0, jnp.float32)
    xlim = jnp.where(wid % 2 == 0, 16, 15)
    lane = lax.iota(jnp.int32, 16)
    plsc.subcore_barrier()

    def compute(j, buf):
        off = j * CHUNK
        for k in range(CHUNK // 16):
            o = off + k * 16
            v0 = blk_v[pl.ds(o, 16)]
            v1 = blk_v[pl.ds(o + 1, 16)]
            v32 = blk_v[pl.ds(o + 32, 16)]
            v1024 = blk_v[pl.ds(o + 1024, 16)]
            pos = lane + o
            zok = (pos & 31) != 31
            yok = ((pos >> 5) & 31) != 31
            xok = (pos >> 10) < xlim
            buf[0, pl.ds(k * 16, 16)] = jnp.where(zok, v0 * NP + v1, 0)
            buf[1, pl.ds(k * 16, 16)] = jnp.where(yok, v0 * NP + v32, 0)
            buf[2, pl.ds(k * 16, 16)] = jnp.where(xok, v0 * NP + v1024, 0)

    def fire(buf, sem):
        for r in range(3):
            pltpu.async_copy(ones_v, c_sh.at[buf.at[r]], sem, add=True)

    def drain(buf, sem):
        for r in range(3):
            pltpu.make_async_copy(ones_v, c_sh.at[buf.at[r]], sem).wait()

    def body(jj, carry):
        compute(2 * jj, ia_v)

        @pl.when(jj > 0)
        def _():
            drain(ib_v, sem_b)

        fire(ia_v, sem_a)
        compute(2 * jj + 1, ib_v)
        drain(ia_v, sem_a)
        fire(ib_v, sem_b)
        return carry

    lax.fori_loop(0, WINDOWS // 2, body, 0)
    drain(ib_v, sem_b)
    plsc.subcore_barrier()
    pltpu.sync_copy(c_sh.at[pl.ds(sid * SLICE, SLICE)],
                    out_hbm.at[pl.ds(cid * CBINS + sid * SLICE, SLICE)])


@functools.cache
def _hist():
    return functools.partial(
        pl.kernel,
        mesh=plsc.VectorSubcoreMesh(core_axis_name="c", subcore_axis_name="s",
                                    num_cores=2, num_subcores=16),
        out_type=jax.ShapeDtypeStruct((2 * CBINS,), jnp.float32),
        scratch_types=[
            pltpu.VMEM((BLK + HALO,), jnp.int32),
            pltpu.VMEM((3, CHUNK), jnp.int32),
            pltpu.VMEM((3, CHUNK), jnp.int32),
            pltpu.VMEM((CHUNK,), jnp.float32),
            pltpu.VMEM_SHARED((CBINS,), jnp.float32),
            pltpu.SemaphoreType.DMA,
            pltpu.SemaphoreType.DMA,
        ],
    )(_hist_body)


def _dmat_body(et_ref, ea_ref, d_ref):
    rows = []
    for j in range(ABLK):
        col = jnp.transpose(ea_ref[pl.ds(j, 1), :])
        x = jnp.abs(et_ref[...] - col.astype(jnp.bfloat16))
        h = D
        while h > 16:
            h //= 2
            x = x[:h] + x[h:]
        dj = jnp.sum(x, axis=0, keepdims=True)
        rows.append(dj)
    d_ref[...] = jnp.concatenate(rows, axis=0)


def _dmat(et, ep):
    return pl.pallas_call(
        _dmat_body,
        grid=(NP // ABLK,),
        in_specs=[
            pl.BlockSpec((D, NP), lambda i: (0, 0)),
            pl.BlockSpec((ABLK, D), lambda i: (i, 0)),
        ],
        out_specs=pl.BlockSpec((ABLK, NP), lambda i: (i, 0)),
        out_shape=jax.ShapeDtypeStruct((NP, NP), jnp.bfloat16),
    )(et, ep)


RBLK = 131072


def _contract_body(c0_ref, c1_ref, d_ref, out_ref, acc_ref):
    i = pl.program_id(0)
    contrib = (c0_ref[...] + c1_ref[...]) * d_ref[...].astype(jnp.float32)

    @pl.when(i == 0)
    def _():
        acc_ref[...] = contrib

    @pl.when(i > 0)
    def _():
        acc_ref[...] = acc_ref[...] + contrib

    @pl.when(i == pl.num_programs(0) - 1)
    def _():
        out_ref[0, 0] = jnp.sum(acc_ref[...])


def _contract(c2f, dmf):
    return pl.pallas_call(
        _contract_body,
        grid=(CBINS // RBLK,),
        in_specs=[
            pl.BlockSpec((RBLK,), lambda i: (i,)),
            pl.BlockSpec((RBLK,), lambda i: (i + CBINS // RBLK,)),
            pl.BlockSpec((RBLK,), lambda i: (i,)),
        ],
        out_specs=pl.BlockSpec((1, 1), lambda i: (0, 0),
                               memory_space=pltpu.SMEM),
        out_shape=jax.ShapeDtypeStruct((1, 1), jnp.float32),
        scratch_shapes=[pltpu.VMEM((RBLK,), jnp.float32)],
    )(c2f, c2f, dmf)


@jax.jit
def kernel(structure, embeddings):
    s = structure
    zeros = jnp.zeros((SLICE,), jnp.float32)
    c2f = _hist()(s.ravel(), zeros)
    ep = jnp.zeros((NP, D), jnp.float32).at[:NV].set(embeddings)
    dm = _dmat(ep.T.astype(jnp.bfloat16), ep)
    total = _contract(c2f, dm.reshape(CBINS))[0, 0]
    n_pairs = s.shape[0] * (s.shape[1] - 1) * s.shape[2] * s.shape[3]
    return total / (3.0 * n_pairs * D)
